# baseline (speedup 1.0000x reference)
import functools

import jax
import jax.numpy as jnp
from jax import lax
from jax.experimental import pallas as pl
from jax.experimental.pallas import tpu as pltpu
from jax.experimental.pallas import tpu_sc as plsc

_V = 1000
_D = 64
_B = 1024
_T = 50
_N = _B * _T


def _table_body(embed_ref, head_ref, out_ref):
    out_ref[...] = lax.dot_general(
        embed_ref[...], head_ref[...],
        dimension_numbers=(((1,), (1,)), ((), ())),
        preferred_element_type=jnp.float32,
    )


def _make_logits_table(embed_table, head_w):
    return pl.pallas_call(
        _table_body,
        out_shape=jax.ShapeDtypeStruct((_V, _V), jnp.float32),
    )(embed_table, head_w)


def _gather_rows(table, idx):
    info = plsc.get_sparse_core_info()
    nc, ns = info.num_cores, info.num_subcores
    nw = nc * ns
    nch, ch = idx.shape[1], idx.shape[2]

    mesh = plsc.VectorSubcoreMesh(core_axis_name="c", subcore_axis_name="s")

    @functools.partial(
        pl.kernel,
        out_type=jax.ShapeDtypeStruct((nw, nch, ch, _V), jnp.float32),
        mesh=mesh,
        compiler_params=pltpu.CompilerParams(use_tc_tiling_on_sc=False),
        scratch_types=[
            pltpu.VMEM((ch,), jnp.int32),
            pltpu.VMEM((ch, _V), jnp.float32),
            pltpu.SemaphoreType.DMA,
        ],
    )
    def k(table_hbm, idx_hbm, out_hbm, idx_v, rows_v, sem):
        wid = lax.axis_index("s") * nc + lax.axis_index("c")

        def body(i, carry):
            pltpu.sync_copy(idx_hbm.at[wid, i], idx_v)
            pltpu.async_copy(table_hbm.at[idx_v], rows_v, sem).wait()
            pltpu.sync_copy(rows_v, out_hbm.at[wid, i])
            return carry

        lax.fori_loop(0, nch, body, 0)

    return k(table, idx)


def kernel(input_ids, embed_table, head_w):
    table = _make_logits_table(embed_table, head_w)
    info = plsc.get_sparse_core_info()
    nw = info.num_cores * info.num_subcores
    ch = 64
    nch = _N // (nw * ch)
    idx = input_ids.reshape(nw, nch, ch).astype(jnp.int32)
    out = _gather_rows(table, idx)
    return out.reshape(_B, _T, _V)


# speedup vs baseline: 1.1361x; 1.1361x over previous
import functools

import jax
import jax.numpy as jnp
from jax import lax
from jax.experimental import pallas as pl
from jax.experimental.pallas import tpu as pltpu
from jax.experimental.pallas import tpu_sc as plsc

_V = 1000
_D = 64
_B = 1024
_T = 50
_N = _B * _T


def _table_body(embed_ref, head_ref, out_ref):
    out_ref[...] = lax.dot_general(
        embed_ref[...], head_ref[...],
        dimension_numbers=(((1,), (1,)), ((), ())),
        preferred_element_type=jnp.float32,
    )


def _make_logits_table(embed_table, head_w):
    return pl.pallas_call(
        _table_body,
        out_shape=jax.ShapeDtypeStruct((_V, _V), jnp.float32),
    )(embed_table, head_w)


def _gather_rows(table, idx):
    info = plsc.get_sparse_core_info()
    nc, ns = info.num_cores, info.num_subcores
    nw = nc * ns
    nch, ch = idx.shape[1], idx.shape[2]
    rows_per_sub = _V // (ns - 8)

    mesh = plsc.VectorSubcoreMesh(core_axis_name="c", subcore_axis_name="s")

    @functools.partial(
        pl.kernel,
        out_type=jax.ShapeDtypeStruct((nw, nch, ch, _V), jnp.float32),
        mesh=mesh,
        compiler_params=pltpu.CompilerParams(use_tc_tiling_on_sc=False),
        scratch_types=[
            pltpu.VMEM((nch, ch), jnp.int32),
            pltpu.VMEM((2, ch, _V), jnp.float32),
            pltpu.VMEM_SHARED((_V, _V), jnp.float32),
            pltpu.SemaphoreType.DMA,
            pltpu.SemaphoreType.DMA,
        ],
    )
    def k(table_hbm, idx_hbm, out_hbm, idx_all, rows2, table_sh, gsem, wsem):
        c = lax.axis_index("c")
        s = lax.axis_index("s")
        wid = s * nc + c

        pltpu.sync_copy(idx_hbm.at[wid], idx_all)

        @pl.when(s < 8)
        def _stage():
            r0 = s * rows_per_sub
            pltpu.sync_copy(table_hbm.at[pl.ds(r0, rows_per_sub)],
                            table_sh.at[pl.ds(r0, rows_per_sub)])

        plsc.subcore_barrier()

        def gather(i, buf):
            pltpu.async_copy(table_sh.at[idx_all.at[i]], rows2.at[buf], gsem)

        def wait_gather(buf):
            pltpu.make_async_copy(table_hbm.at[pl.ds(0, ch)], rows2.at[buf],
                                  gsem).wait()

        def write(i, buf):
            pltpu.async_copy(rows2.at[buf], out_hbm.at[wid, i], wsem)

        def wait_write(i, buf):
            pltpu.make_async_copy(rows2.at[buf], out_hbm.at[wid, i], wsem).wait()

        gather(0, 0)
        gather(1, 1)
        wait_gather(0)
        write(0, 0)

        def body(i, carry):
            p = lax.rem(i, 2)
            q = 1 - p
            wait_write(i - 1, q)
            gather(i + 1, q)
            wait_gather(p)
            write(i, p)
            return carry

        lax.fori_loop(1, nch - 1, body, 0)

        pl_last = lax.rem(nch - 1, 2)
        wait_write(nch - 2, 1 - pl_last)
        wait_gather(pl_last)
        write(nch - 1, pl_last)
        wait_write(nch - 1, pl_last)

    return k(table, idx)


def kernel(input_ids, embed_table, head_w):
    table = _make_logits_table(embed_table, head_w)
    info = plsc.get_sparse_core_info()
    nw = info.num_cores * info.num_subcores
    ch = 32
    nch = _N // (nw * ch)
    idx = input_ids.reshape(nw, nch, ch).astype(jnp.int32)
    out = _gather_rows(table, idx)
    return out.reshape(_B, _T, _V)
